# baseline (speedup 1.0000x reference)
import functools
import math

import jax
import jax.numpy as jnp
from jax import lax
from jax.experimental import pallas as pl
from jax.experimental.pallas import tpu as pltpu
from jax.experimental.pallas import tpu_sc as plsc

ROWS = 4096
SEQ = 50
D = 128
SCALE = math.sqrt(D)

NC = 2
NS = 16
NW = NC * NS
LANES = 16

BAND = ROWS // NW
NBUF = 2
GS = 4
QS = BAND // GS


def _body(tok_hbm, table_hbm, out_hbm, idx_v, in_v, out_v, gsem, wsem):
    wid = lax.axis_index("s") * NC + lax.axis_index("c")
    base = wid * BAND

    pltpu.sync_copy(tok_hbm.at[:, pl.ds(base, BAND)], idx_v)

    def gather_start(t, b):
        for q in range(GS):
            pltpu.async_copy(table_hbm.at[idx_v.at[t, pl.ds(q * QS, QS)]],
                             in_v[b].at[pl.ds(q * QS, QS)], gsem[b])

    def gather_wait(t, b):
        for q in range(GS):
            pltpu.make_async_copy(table_hbm.at[idx_v.at[t, pl.ds(q * QS, QS)]],
                                  in_v[b].at[pl.ds(q * QS, QS)],
                                  gsem[b]).wait()

    def write_start(t, b):
        pltpu.async_copy(out_v[b], out_hbm.at[t, pl.ds(base, BAND)], wsem[b])

    def write_wait(t, b):
        pltpu.make_async_copy(out_v[b], out_hbm.at[t, pl.ds(base, BAND)],
                              wsem[b]).wait()

    def scale(b):
        @pl.loop(0, BAND, unroll=2)
        def _row(r):
            for k in range(D // LANES):
                sl = pl.ds(k * LANES, LANES)
                out_v[b][r, sl] = in_v[b][r, sl] * SCALE

    for b in range(NBUF):
        gather_start(b, b)

    @pl.loop(0, SEQ, step=NBUF)
    def _grp(j):
        for b in range(NBUF):
            t = j + b
            gather_wait(t, b)

            @pl.when(t >= NBUF)
            def _():
                write_wait(t - NBUF, b)

            scale(b)

            @pl.when(t + NBUF < SEQ)
            def _():
                gather_start(t + NBUF, b)

            write_start(t, b)

    for b in range(NBUF):
        write_wait(SEQ - NBUF + b, b)


@jax.jit
def _embed(tokens_t, table):
    mesh = plsc.VectorSubcoreMesh(
        core_axis_name="c", subcore_axis_name="s",
        num_cores=NC, num_subcores=NS,
    )
    kern = pl.kernel(
        _body,
        out_type=jax.ShapeDtypeStruct((SEQ, ROWS, D), jnp.float32),
        mesh=mesh,
        scratch_types=[
            pltpu.VMEM((SEQ, BAND), jnp.int32),
            [pltpu.VMEM((BAND, D), jnp.float32) for _ in range(NBUF)],
            [pltpu.VMEM((BAND, D), jnp.float32) for _ in range(NBUF)],
            [pltpu.SemaphoreType.DMA for _ in range(NBUF)],
            [pltpu.SemaphoreType.DMA for _ in range(NBUF)],
        ],
    )
    return kern(tokens_t, table)


def kernel(tokens, table):
    out = _embed(tokens.astype(jnp.int32).T, table)
    return jnp.swapaxes(out, 0, 1)


# speedup vs baseline: 1.0007x; 1.0007x over previous
import functools
import math

import jax
import jax.numpy as jnp
from jax import lax
from jax.experimental import pallas as pl
from jax.experimental.pallas import tpu as pltpu
from jax.experimental.pallas import tpu_sc as plsc

ROWS = 4096
SEQ = 50
D = 128
SCALE = math.sqrt(D)

NC = 2
NS = 16
NW = NC * NS
LANES = 16

B_TOTAL = ROWS * SEQ
B_PER_W = B_TOTAL // NW
CHUNK = 128
N_CHUNKS = B_PER_W // CHUNK
NBUF = 2
GS = 4
QS = CHUNK // GS


def _body(tok_hbm, table_hbm, out_hbm, idx_v, in_v, out_v, gsem, wsem):
    wid = lax.axis_index("s") * NC + lax.axis_index("c")
    base = wid * B_PER_W

    pltpu.sync_copy(tok_hbm.at[pl.ds(base, B_PER_W)], idx_v)

    def gather_start(c, b):
        for q in range(GS):
            pltpu.async_copy(table_hbm.at[idx_v.at[pl.ds(c * CHUNK + q * QS,
                                                         QS)]],
                             in_v[b].at[pl.ds(q * QS, QS)], gsem[b])

    def gather_wait(c, b):
        for q in range(GS):
            pltpu.make_async_copy(
                table_hbm.at[idx_v.at[pl.ds(c * CHUNK + q * QS, QS)]],
                in_v[b].at[pl.ds(q * QS, QS)], gsem[b]).wait()

    def write_start(c, b):
        pltpu.async_copy(out_v[b],
                         out_hbm.at[pl.ds(base + c * CHUNK, CHUNK)], wsem[b])

    def write_wait(c, b):
        pltpu.make_async_copy(out_v[b],
                              out_hbm.at[pl.ds(base + c * CHUNK, CHUNK)],
                              wsem[b]).wait()

    def scale(b):
        @pl.loop(0, CHUNK, unroll=2)
        def _row(r):
            for k in range(D // LANES):
                sl = pl.ds(k * LANES, LANES)
                out_v[b][r, sl] = in_v[b][r, sl] * SCALE

    for b in range(NBUF):
        gather_start(b, b)

    @pl.loop(0, N_CHUNKS, step=NBUF)
    def _grp(j):
        for b in range(NBUF):
            c = j + b
            gather_wait(c, b)

            @pl.when(c >= NBUF)
            def _():
                write_wait(c - NBUF, b)

            scale(b)

            @pl.when(c + NBUF < N_CHUNKS)
            def _():
                gather_start(c + NBUF, b)

            write_start(c, b)

    for b in range(NBUF):
        write_wait(N_CHUNKS - NBUF + b, b)


@jax.jit
def _embed(tokens_flat, table):
    mesh = plsc.VectorSubcoreMesh(
        core_axis_name="c", subcore_axis_name="s",
        num_cores=NC, num_subcores=NS,
    )
    kern = pl.kernel(
        _body,
        out_type=jax.ShapeDtypeStruct((B_TOTAL, D), jnp.float32),
        mesh=mesh,
        scratch_types=[
            pltpu.VMEM((B_PER_W,), jnp.int32),
            [pltpu.VMEM((CHUNK, D), jnp.float32) for _ in range(NBUF)],
            [pltpu.VMEM((CHUNK, D), jnp.float32) for _ in range(NBUF)],
            [pltpu.SemaphoreType.DMA for _ in range(NBUF)],
            [pltpu.SemaphoreType.DMA for _ in range(NBUF)],
        ],
    )
    return kern(tokens_flat, table)


def kernel(tokens, table):
    tok_flat = tokens.astype(jnp.int32).T.reshape(B_TOTAL)
    out = _embed(tok_flat, table)
    return jnp.swapaxes(out.reshape(SEQ, ROWS, D), 0, 1)


# speedup vs baseline: 1.0019x; 1.0012x over previous
import functools
import math

import jax
import jax.numpy as jnp
from jax import lax
from jax.experimental import pallas as pl
from jax.experimental.pallas import tpu as pltpu
from jax.experimental.pallas import tpu_sc as plsc

ROWS = 4096
SEQ = 50
D = 128
SCALE = math.sqrt(D)

NC = 2
NS = 16
NW = NC * NS
LANES = 16

B_TOTAL = ROWS * SEQ
B_PER_W = B_TOTAL // NW
CHUNK = 128
N_CHUNKS = B_PER_W // CHUNK
NBUF = 2
GS = 4
QS = CHUNK // GS


def _body(tok_hbm, table_hbm, out_hbm, idx_v, in_v, out_v, gsem, wsem):
    wid = lax.axis_index("s") * NC + lax.axis_index("c")
    base = wid * B_PER_W

    pltpu.sync_copy(tok_hbm.at[pl.ds(base, B_PER_W)], idx_v)

    def gather_start(c, b):
        for q in range(GS):
            pltpu.async_copy(table_hbm.at[idx_v.at[pl.ds(c * CHUNK + q * QS,
                                                         QS)]],
                             in_v[b].at[pl.ds(q * QS, QS)], gsem[b])

    def gather_wait(c, b):
        for q in range(GS):
            pltpu.make_async_copy(
                table_hbm.at[idx_v.at[pl.ds(c * CHUNK + q * QS, QS)]],
                in_v[b].at[pl.ds(q * QS, QS)], gsem[b]).wait()

    def write_start(c, b):
        pltpu.async_copy(out_v[b],
                         out_hbm.at[pl.ds(base + c * CHUNK, CHUNK)], wsem[b])

    def write_wait(c, b):
        pltpu.make_async_copy(out_v[b],
                              out_hbm.at[pl.ds(base + c * CHUNK, CHUNK)],
                              wsem[b]).wait()

    def scale(b):
        @pl.loop(0, CHUNK, unroll=8)
        def _row(r):
            for k in range(D // LANES):
                sl = pl.ds(k * LANES, LANES)
                out_v[b][r, sl] = in_v[b][r, sl] * SCALE

    for b in range(NBUF):
        gather_start(b, b)

    @pl.loop(0, N_CHUNKS, step=NBUF)
    def _grp(j):
        for b in range(NBUF):
            c = j + b
            gather_wait(c, b)

            @pl.when(c >= NBUF)
            def _():
                write_wait(c - NBUF, b)

            scale(b)

            @pl.when(c + NBUF < N_CHUNKS)
            def _():
                gather_start(c + NBUF, b)

            write_start(c, b)

    for b in range(NBUF):
        write_wait(N_CHUNKS - NBUF + b, b)


@jax.jit
def _embed(tokens_flat, table):
    mesh = plsc.VectorSubcoreMesh(
        core_axis_name="c", subcore_axis_name="s",
        num_cores=NC, num_subcores=NS,
    )
    kern = pl.kernel(
        _body,
        out_type=jax.ShapeDtypeStruct((B_TOTAL, D), jnp.float32),
        mesh=mesh,
        scratch_types=[
            pltpu.VMEM((B_PER_W,), jnp.int32),
            [pltpu.VMEM((CHUNK, D), jnp.float32) for _ in range(NBUF)],
            [pltpu.VMEM((CHUNK, D), jnp.float32) for _ in range(NBUF)],
            [pltpu.SemaphoreType.DMA for _ in range(NBUF)],
            [pltpu.SemaphoreType.DMA for _ in range(NBUF)],
        ],
    )
    return kern(tokens_flat, table)


def kernel(tokens, table):
    tok_flat = tokens.astype(jnp.int32).T.reshape(B_TOTAL)
    out = _embed(tok_flat, table)
    return jnp.swapaxes(out.reshape(SEQ, ROWS, D), 0, 1)


# speedup vs baseline: 1.5725x; 1.5695x over previous
import functools
import math

import jax
import jax.numpy as jnp
from jax import lax
from jax.experimental import pallas as pl
from jax.experimental.pallas import tpu as pltpu
from jax.experimental.pallas import tpu_sc as plsc

ROWS = 4096
SEQ = 50
D = 128
SCALE = math.sqrt(D)

NC = 2
NS = 16
NW = NC * NS
LANES = 16

R_PER_W = ROWS // NW
RCHUNK = 4
N_CHUNKS = R_PER_W // RCHUNK
NBUF = 2


def _body(tok_hbm, table_hbm, out_hbm, idx_v, in_v, out_v, gsem, wsem):
    wid = lax.axis_index("s") * NC + lax.axis_index("c")
    base = wid * R_PER_W

    pltpu.sync_copy(tok_hbm.at[pl.ds(base, R_PER_W)], idx_v)

    def gather_start(c, b):
        for r in range(RCHUNK):
            pltpu.async_copy(table_hbm.at[idx_v.at[c * RCHUNK + r]],
                             in_v[b].at[r], gsem[b])

    def gather_wait(c, b):
        for r in range(RCHUNK):
            pltpu.make_async_copy(table_hbm.at[idx_v.at[c * RCHUNK + r]],
                                  in_v[b].at[r], gsem[b]).wait()

    def write_start(c, b):
        pltpu.async_copy(
            out_v[b], out_hbm.at[pl.ds(base + c * RCHUNK, RCHUNK)], wsem[b])

    def write_wait(c, b):
        pltpu.make_async_copy(
            out_v[b], out_hbm.at[pl.ds(base + c * RCHUNK, RCHUNK)],
            wsem[b]).wait()

    def scale(b):
        @pl.loop(0, SEQ, unroll=2)
        def _tok(t):
            for r in range(RCHUNK):
                for k in range(D // LANES):
                    sl = pl.ds(k * LANES, LANES)
                    out_v[b][r, t, sl] = in_v[b][r, t, sl] * SCALE

    for b in range(NBUF):
        gather_start(b, b)

    @pl.loop(0, N_CHUNKS, step=NBUF)
    def _grp(j):
        for b in range(NBUF):
            c = j + b
            gather_wait(c, b)

            @pl.when(c >= NBUF)
            def _():
                write_wait(c - NBUF, b)

            scale(b)

            @pl.when(c + NBUF < N_CHUNKS)
            def _():
                gather_start(c + NBUF, b)

            write_start(c, b)

    for b in range(NBUF):
        write_wait(N_CHUNKS - NBUF + b, b)


@jax.jit
def _embed(tokens, table):
    mesh = plsc.VectorSubcoreMesh(
        core_axis_name="c", subcore_axis_name="s",
        num_cores=NC, num_subcores=NS,
    )
    kern = pl.kernel(
        _body,
        out_type=jax.ShapeDtypeStruct((ROWS, SEQ, D), jnp.float32),
        mesh=mesh,
        scratch_types=[
            pltpu.VMEM((R_PER_W, SEQ), jnp.int32),
            [pltpu.VMEM((RCHUNK, SEQ, D), jnp.float32) for _ in range(NBUF)],
            [pltpu.VMEM((RCHUNK, SEQ, D), jnp.float32) for _ in range(NBUF)],
            [pltpu.SemaphoreType.DMA for _ in range(NBUF)],
            [pltpu.SemaphoreType.DMA for _ in range(NBUF)],
        ],
    )
    return kern(tokens, table)


def kernel(tokens, table):
    return _embed(tokens.astype(jnp.int32), table)
